# Initial kernel scaffold; baseline (speedup 1.0000x reference)
import jax
import jax.numpy as jnp
from jax.experimental import pallas as pl


def kernel(category_emb, user_emb, edge_index, edge_type, interact_mat, weight):
    raise NotImplementedError("write your pallas kernel here")


# speedup vs baseline: 6.5270x; 6.5270x over previous
import functools

import jax
import jax.numpy as jnp
from jax import lax
from jax.experimental import pallas as pl
from jax.experimental.pallas import tpu as pltpu
from jax.experimental.pallas import tpu_sc as plsc

_NC = 10000
_NU = 4096
_D = 128
_E = 320000
_R = 15
_RP = 16

_NW = 32
_PER_W = _E // _NW
_B = 80
_NCH = _PER_W // _B
_NZ = 10240
_STRIPE = _NZ // 16


def _tables_body(emb_ref, w_ref, a_ref, t_ref):
    k = pl.program_id(0)

    @pl.when(k == 0)
    def _():
        e2 = emb_ref[...] * emb_ref[...]
        w2 = w_ref[...] * w_ref[...]
        a_ref[...] = lax.dot_general(
            e2, w2, (((1,), (1,)), ((), ())),
            preferred_element_type=jnp.float32)

    wrow = w_ref[pl.ds(k, 1), :]
    t_ref[...] = (emb_ref[...] * wrow)[None]


def _make_tables(emb, wpad):
    return pl.pallas_call(
        _tables_body,
        grid=(_R,),
        in_specs=[
            pl.BlockSpec((_NC, _D), lambda k: (0, 0)),
            pl.BlockSpec((_RP, _D), lambda k: (0, 0)),
        ],
        out_specs=[
            pl.BlockSpec((_NC, _RP), lambda k: (0, 0)),
            pl.BlockSpec((1, _NC, _D), lambda k: (k, 0, 0)),
        ],
        out_shape=[
            jax.ShapeDtypeStruct((_NC, _RP), jnp.float32),
            jax.ShapeDtypeStruct((_R, _NC, _D), jnp.float32),
        ],
    )(emb, wpad)


def _user_body(im_ref, emb_ref, ue_ref, w_ref, out_ref):
    agg = lax.dot_general(
        im_ref[...], emb_ref[...], (((1,), (0,)), ((), ())),
        preferred_element_type=jnp.float32)
    logits = lax.dot_general(
        ue_ref[...], w_ref[...], (((1,), (1,)), ((), ())),
        preferred_element_type=jnp.float32)
    col = lax.broadcasted_iota(jnp.int32, logits.shape, 1)
    logits = jnp.where(col < _R, logits, -1e30)
    m = jnp.max(logits, axis=-1, keepdims=True)
    ex = jnp.exp(logits - m)
    score = ex / jnp.sum(ex, axis=-1, keepdims=True)
    corr = lax.dot_general(
        score, w_ref[...], (((1,), (0,)), ((), ())),
        preferred_element_type=jnp.float32)
    out_ref[...] = agg + corr * agg


def _make_user(interact_mat, emb, user_emb, wpad):
    bu = 256
    return pl.pallas_call(
        _user_body,
        grid=(_NU // bu,),
        in_specs=[
            pl.BlockSpec((bu, _NC), lambda i: (i, 0)),
            pl.BlockSpec((_NC, _D), lambda i: (0, 0)),
            pl.BlockSpec((bu, _D), lambda i: (i, 0)),
            pl.BlockSpec((_RP, _D), lambda i: (0, 0)),
        ],
        out_specs=pl.BlockSpec((bu, _D), lambda i: (i, 0)),
        out_shape=jax.ShapeDtypeStruct((_NU, _D), jnp.float32),
    )(interact_mat, emb, user_emb, wpad)


def _combine_body(in_ref, out_ref):
    out_ref[...] = in_ref[0] + in_ref[1]


def _make_combine(catpart):
    br = 1000
    return pl.pallas_call(
        _combine_body,
        grid=(_NC // br,),
        in_specs=[pl.BlockSpec((2, br, _D), lambda i: (0, i, 0))],
        out_specs=pl.BlockSpec((br, _D), lambda i: (i, 0)),
        out_shape=jax.ShapeDtypeStruct((_NC, _D), jnp.float32),
    )(catpart)


@functools.partial(
    pl.kernel,
    mesh=plsc.VectorSubcoreMesh(core_axis_name="c", subcore_axis_name="s"),
    compiler_params=pltpu.CompilerParams(needs_layout_passes=False),
    out_type=[
        jax.ShapeDtypeStruct((_E,), jnp.float32),
        jax.ShapeDtypeStruct((2, _NZ), jnp.float32),
    ],
    scratch_types=[
        pltpu.VMEM((_B,), jnp.int32),
        pltpu.VMEM((_B,), jnp.int32),
        pltpu.VMEM((_B,), jnp.int32),
        pltpu.VMEM((_B,), jnp.int32),
        pltpu.VMEM((_B,), jnp.int32),
        pltpu.VMEM((_B,), jnp.float32),
        pltpu.VMEM((_B,), jnp.float32),
        pltpu.VMEM((_B,), jnp.float32),
        pltpu.VMEM((_STRIPE,), jnp.float32),
        pltpu.VMEM_SHARED((_NZ,), jnp.float32),
        pltpu.SemaphoreType.DMA,
    ],
)
def _s1(head_hbm, tail_hbm, ktype_hbm, aflat_hbm, exps_hbm, zpart_hbm,
        hv, tv, kv, ih, it, ah, at, es, zb, zs, sem):
    c = lax.axis_index("c")
    s = lax.axis_index("s")
    wid = s * 2 + c

    def _zfill(i, carry):
        zb[pl.ds(i * 16, 16)] = jnp.zeros((16,), jnp.float32)
        return carry
    lax.fori_loop(0, _STRIPE // 16, _zfill, 0)
    pltpu.sync_copy(zb, zs.at[pl.ds(s * _STRIPE, _STRIPE)])
    plsc.subcore_barrier()

    def _chunk(g, carry):
        base = wid * _PER_W + g * _B
        pltpu.sync_copy(head_hbm.at[pl.ds(base, _B)], hv)
        pltpu.sync_copy(tail_hbm.at[pl.ds(base, _B)], tv)
        pltpu.sync_copy(ktype_hbm.at[pl.ds(base, _B)], kv)

        def _idx(i, cy):
            sl = pl.ds(i * 16, 16)
            k16 = kv[sl] - 1
            ih[sl] = hv[sl] * _RP + k16
            it[sl] = tv[sl] * _RP + k16
            return cy
        lax.fori_loop(0, _B // 16, _idx, 0)

        cp1 = pltpu.async_copy(aflat_hbm.at[ih], ah, sem)
        cp1.wait()
        cp2 = pltpu.async_copy(aflat_hbm.at[it], at, sem)
        cp2.wait()

        def _exp(i, cy):
            sl = pl.ds(i * 16, 16)
            es[sl] = jnp.exp(ah[sl] * at[sl])
            return cy
        lax.fori_loop(0, _B // 16, _exp, 0)

        pltpu.sync_copy(es, exps_hbm.at[pl.ds(base, _B)])
        pltpu.sync_copy(es, zs.at[hv], add=True)
        return carry

    lax.fori_loop(0, _NCH, _chunk, 0)
    plsc.subcore_barrier()

    pltpu.sync_copy(zs.at[pl.ds(s * _STRIPE, _STRIPE)], zb)
    pltpu.sync_copy(zb, zpart_hbm.at[c, pl.ds(s * _STRIPE, _STRIPE)])


@functools.partial(
    pl.kernel,
    mesh=plsc.VectorSubcoreMesh(core_axis_name="c", subcore_axis_name="s"),
    compiler_params=pltpu.CompilerParams(needs_layout_passes=False),
    out_type=[
        jax.ShapeDtypeStruct((2, _NZ, _D), jnp.float32),
    ],
    scratch_types=[
        pltpu.VMEM((_B,), jnp.int32),
        pltpu.VMEM((_B,), jnp.int32),
        pltpu.VMEM((_B,), jnp.int32),
        pltpu.VMEM((_B,), jnp.int32),
        pltpu.VMEM((_B,), jnp.float32),
        pltpu.VMEM((_B,), jnp.float32),
        pltpu.VMEM((_B, _D), jnp.float32),
        pltpu.VMEM((_NZ,), jnp.float32),
        pltpu.VMEM((_NZ,), jnp.float32),
        pltpu.VMEM_SHARED((_NZ, _D), jnp.float32),
        pltpu.SemaphoreType.DMA,
    ],
)
def _s2(head_hbm, tail_hbm, ktype_hbm, exps_hbm, zpart_hbm, tflat_hbm,
        cat_hbm, hv, tv, kv, idx, ev, wv, rows, zv, zv2, cat_s, sem):
    c = lax.axis_index("c")
    s = lax.axis_index("s")
    wid = s * 2 + c

    def _zrow(i, carry):
        for j in range(_D // 16):
            rows[i, pl.ds(j * 16, 16)] = jnp.zeros((16,), jnp.float32)
        return carry
    lax.fori_loop(0, _B, _zrow, 0)

    def _zcat(i, carry):
        pltpu.sync_copy(rows, cat_s.at[pl.ds(s * _STRIPE + i * _B, _B)])
        return carry
    lax.fori_loop(0, _STRIPE // _B, _zcat, 0)
    plsc.subcore_barrier()

    pltpu.sync_copy(zpart_hbm.at[0], zv)
    pltpu.sync_copy(zpart_hbm.at[1], zv2)

    def _zadd(i, carry):
        sl = pl.ds(i * 16, 16)
        zv[sl] = zv[sl] + zv2[sl]
        return carry
    lax.fori_loop(0, _NZ // 16, _zadd, 0)

    def _chunk(g, carry):
        base = wid * _PER_W + g * _B
        pltpu.sync_copy(head_hbm.at[pl.ds(base, _B)], hv)
        pltpu.sync_copy(tail_hbm.at[pl.ds(base, _B)], tv)
        pltpu.sync_copy(ktype_hbm.at[pl.ds(base, _B)], kv)
        pltpu.sync_copy(exps_hbm.at[pl.ds(base, _B)], ev)

        def _wts(i, cy):
            sl = pl.ds(i * 16, 16)
            k16 = kv[sl] - 1
            idx[sl] = k16 * _NC + tv[sl]
            zg = plsc.load_gather(zv, [hv[sl]])
            wv[sl] = ev[sl] / (zg + 1e-16)
            return cy
        lax.fori_loop(0, _B // 16, _wts, 0)

        pltpu.async_copy(tflat_hbm.at[idx], rows, sem).wait()

        def _scale(e, cy):
            wsp = plsc.load_gather(wv, [jnp.full((16,), e, jnp.int32)])
            for j in range(_D // 16):
                sl = pl.ds(j * 16, 16)
                rows[e, sl] = rows[e, sl] * wsp
            return cy
        lax.fori_loop(0, _B, _scale, 0)

        pltpu.sync_copy(rows, cat_s.at[hv], add=True)
        return carry

    lax.fori_loop(0, _NCH, _chunk, 0)
    plsc.subcore_barrier()

    def _out(i, carry):
        r0 = s * _STRIPE + i * _B
        pltpu.sync_copy(cat_s.at[pl.ds(r0, _B)], rows)
        pltpu.sync_copy(rows, cat_hbm.at[c, pl.ds(r0, _B)])
        return carry
    lax.fori_loop(0, _STRIPE // _B, _out, 0)


def kernel(category_emb, user_emb, edge_index, edge_type, interact_mat, weight):
    head = edge_index[0].astype(jnp.int32)
    tail = edge_index[1].astype(jnp.int32)
    ktype = edge_type.astype(jnp.int32)
    wpad = jnp.pad(weight, ((0, _RP - _R), (0, 0)))

    a, t = _make_tables(category_emb, wpad)
    aflat = a.reshape(_NC * _RP)
    tflat = t.reshape(_R * _NC, _D)

    exps, zpart = _s1(head, tail, ktype, aflat)
    (catpart,) = _s2(head, tail, ktype, exps, zpart, tflat)
    category_agg = _make_combine(catpart)

    user_agg = _make_user(interact_mat, category_emb, user_emb, wpad)
    return (category_agg, user_agg)


# speedup vs baseline: 7.6669x; 1.1747x over previous
import functools

import jax
import jax.numpy as jnp
from jax import lax
from jax.experimental import pallas as pl
from jax.experimental.pallas import tpu as pltpu
from jax.experimental.pallas import tpu_sc as plsc

_NC = 10000
_NU = 4096
_D = 128
_E = 320000
_R = 15
_RP = 16

_NW = 32
_BP = 128
_PW = 10240
_EP = _NW * _PW
_NS1 = _PW // _BP
_SC2 = 10
_NS2 = _PW // (_SC2 * _BP)
_NZ = 10240
_STRIPE = _NZ // 16


def _tables_body(emb_ref, w_ref, a_ref, t_ref):
    k = pl.program_id(0)

    @pl.when(k == 0)
    def _():
        e2 = emb_ref[...] * emb_ref[...]
        w2 = w_ref[...] * w_ref[...]
        a_ref[...] = lax.dot_general(
            e2, w2, (((1,), (1,)), ((), ())),
            preferred_element_type=jnp.float32)

    wrow = w_ref[pl.ds(k, 1), :]
    t_ref[...] = (emb_ref[...] * wrow)[None]


def _make_tables(emb, wpad):
    return pl.pallas_call(
        _tables_body,
        grid=(_R,),
        in_specs=[
            pl.BlockSpec((_NC, _D), lambda k: (0, 0)),
            pl.BlockSpec((_RP, _D), lambda k: (0, 0)),
        ],
        out_specs=[
            pl.BlockSpec((_NC, _RP), lambda k: (0, 0)),
            pl.BlockSpec((1, _NC, _D), lambda k: (k, 0, 0)),
        ],
        out_shape=[
            jax.ShapeDtypeStruct((_NC, _RP), jnp.float32),
            jax.ShapeDtypeStruct((_R, _NC, _D), jnp.float32),
        ],
    )(emb, wpad)


def _user_body(im_ref, emb_ref, ue_ref, w_ref, out_ref):
    agg = lax.dot_general(
        im_ref[...], emb_ref[...], (((1,), (0,)), ((), ())),
        preferred_element_type=jnp.float32)
    logits = lax.dot_general(
        ue_ref[...], w_ref[...], (((1,), (1,)), ((), ())),
        preferred_element_type=jnp.float32)
    col = lax.broadcasted_iota(jnp.int32, logits.shape, 1)
    logits = jnp.where(col < _R, logits, -1e30)
    m = jnp.max(logits, axis=-1, keepdims=True)
    ex = jnp.exp(logits - m)
    score = ex / jnp.sum(ex, axis=-1, keepdims=True)
    corr = lax.dot_general(
        score, w_ref[...], (((1,), (0,)), ((), ())),
        preferred_element_type=jnp.float32)
    out_ref[...] = agg + corr * agg


def _make_user(interact_mat, emb, user_emb, wpad):
    bu = 256
    return pl.pallas_call(
        _user_body,
        grid=(_NU // bu,),
        in_specs=[
            pl.BlockSpec((bu, _NC), lambda i: (i, 0)),
            pl.BlockSpec((_NC, _D), lambda i: (0, 0)),
            pl.BlockSpec((bu, _D), lambda i: (i, 0)),
            pl.BlockSpec((_RP, _D), lambda i: (0, 0)),
        ],
        out_specs=pl.BlockSpec((bu, _D), lambda i: (i, 0)),
        out_shape=jax.ShapeDtypeStruct((_NU, _D), jnp.float32),
    )(interact_mat, emb, user_emb, wpad)


def _combine_body(in_ref, out_ref):
    out_ref[...] = in_ref[0] + in_ref[1]


def _make_combine(catpart):
    br = 1000
    return pl.pallas_call(
        _combine_body,
        grid=(_NC // br,),
        in_specs=[pl.BlockSpec((2, br, _D), lambda i: (0, i, 0))],
        out_specs=pl.BlockSpec((br, _D), lambda i: (i, 0)),
        out_shape=jax.ShapeDtypeStruct((_NC, _D), jnp.float32),
    )(catpart)


@functools.partial(
    pl.kernel,
    mesh=plsc.VectorSubcoreMesh(core_axis_name="c", subcore_axis_name="s"),
    compiler_params=pltpu.CompilerParams(needs_layout_passes=False),
    out_type=[
        jax.ShapeDtypeStruct((_NW, _NS1, _BP), jnp.float32),
        jax.ShapeDtypeStruct((2, _NZ), jnp.float32),
    ],
    scratch_types=[
        pltpu.VMEM((_NS1, _BP), jnp.int32),
        pltpu.VMEM((_NS1, _BP), jnp.int32),
        pltpu.VMEM((_NS1, _BP), jnp.int32),
        pltpu.VMEM((_NS1, _BP), jnp.int32),
        pltpu.VMEM((_NS1, _BP), jnp.int32),
        pltpu.VMEM((_NS1, _BP), jnp.float32),
        pltpu.VMEM((_NS1, _BP), jnp.float32),
        pltpu.VMEM((_STRIPE,), jnp.float32),
        pltpu.VMEM_SHARED((_NZ,), jnp.float32),
        pltpu.SemaphoreType.DMA,
        pltpu.SemaphoreType.DMA,
    ],
)
def _s1(head_hbm, tail_hbm, ktype_hbm, aflat_hbm, exps_hbm, zpart_hbm,
        hv, tv, kv, ih, it, ah, at, zb, zs, gsem, ssem):
    c = lax.axis_index("c")
    s = lax.axis_index("s")
    wid = s * 2 + c

    def _zfill(i, carry):
        zb[pl.ds(i * 16, 16)] = jnp.zeros((16,), jnp.float32)
        return carry
    lax.fori_loop(0, _STRIPE // 16, _zfill, 0)
    pltpu.sync_copy(zb, zs.at[pl.ds(s * _STRIPE, _STRIPE)])
    plsc.subcore_barrier()

    pltpu.sync_copy(head_hbm.at[wid], hv)
    pltpu.sync_copy(tail_hbm.at[wid], tv)
    pltpu.sync_copy(ktype_hbm.at[wid], kv)

    def _idx(j, cy):
        for i in range(_BP // 16):
            sl = pl.ds(i * 16, 16)
            k16 = kv[j, sl] - 1
            ih[j, sl] = hv[j, sl] * _RP + k16
            it[j, sl] = tv[j, sl] * _RP + k16
        return cy
    lax.fori_loop(0, _NS1, _idx, 0)

    def _fire(j, cy):
        pltpu.async_copy(aflat_hbm.at[ih.at[j]], ah.at[j], gsem)
        pltpu.async_copy(aflat_hbm.at[it.at[j]], at.at[j], gsem)
        return cy
    lax.fori_loop(0, _NS1, _fire, 0)
    pltpu.make_async_copy(exps_hbm.at[0], ah, gsem).wait()
    pltpu.make_async_copy(exps_hbm.at[0], at, gsem).wait()

    def _exp(j, cy):
        for i in range(_BP // 16):
            sl = pl.ds(i * 16, 16)
            ah[j, sl] = jnp.exp(ah[j, sl] * at[j, sl])
        return cy
    lax.fori_loop(0, _NS1, _exp, 0)

    pltpu.sync_copy(ah, exps_hbm.at[wid])

    def _scat(j, cy):
        pltpu.async_copy(ah.at[j], zs.at[hv.at[j]], ssem, add=True)
        return cy
    lax.fori_loop(0, _NS1, _scat, 0)
    pltpu.make_async_copy(exps_hbm.at[0], ah, ssem).wait()
    plsc.subcore_barrier()

    pltpu.sync_copy(zs.at[pl.ds(s * _STRIPE, _STRIPE)], zb)
    pltpu.sync_copy(zb, zpart_hbm.at[c, pl.ds(s * _STRIPE, _STRIPE)])


@functools.partial(
    pl.kernel,
    mesh=plsc.VectorSubcoreMesh(core_axis_name="c", subcore_axis_name="s"),
    compiler_params=pltpu.CompilerParams(needs_layout_passes=False),
    out_type=[
        jax.ShapeDtypeStruct((2, _NZ, _D), jnp.float32),
    ],
    scratch_types=[
        pltpu.VMEM((_NS2, _BP), jnp.int32),
        pltpu.VMEM((_NS2, _BP), jnp.int32),
        pltpu.VMEM((_NS2, _BP), jnp.int32),
        pltpu.VMEM((_NS2, _BP), jnp.float32),
        pltpu.VMEM((512,), jnp.float32),
        pltpu.VMEM((_NZ,), jnp.float32),
        pltpu.VMEM((2, _BP, _D), jnp.float32),
        pltpu.VMEM_SHARED((_NZ, _D), jnp.float32),
        pltpu.SemaphoreType.DMA,
        pltpu.SemaphoreType.DMA,
    ],
)
def _s2(head_hbm, tail_hbm, ktype_hbm, exps_hbm, zpart_hbm, tflat_hbm,
        cat_hbm, hv, kv, idx, wv, ztmp, zv, rows, cat_s, gsem, ssem):
    c = lax.axis_index("c")
    s = lax.axis_index("s")
    wid = s * 2 + c

    def _zrow(i, carry):
        for j in range(_D // 16):
            rows[0, i, pl.ds(j * 16, 16)] = jnp.zeros((16,), jnp.float32)
        return carry
    lax.fori_loop(0, _BP, _zrow, 0)

    def _zcat(i, carry):
        pltpu.sync_copy(rows.at[0], cat_s.at[pl.ds(s * _STRIPE + i * _BP, _BP)])
        return carry
    lax.fori_loop(0, _STRIPE // _BP, _zcat, 0)
    plsc.subcore_barrier()

    pltpu.sync_copy(zpart_hbm.at[0], zv)

    def _zld(i, carry):
        pltpu.sync_copy(zpart_hbm.at[1, pl.ds(i * 512, 512)], ztmp)

        def _za(g, c2):
            zv[pl.ds(i * 512 + g * 16, 16)] = (
                zv[pl.ds(i * 512 + g * 16, 16)] + ztmp[pl.ds(g * 16, 16)])
            return c2
        lax.fori_loop(0, 512 // 16, _za, 0)
        return carry
    lax.fori_loop(0, _NZ // 512, _zld, 0)

    def _super(sc, cy0):
        pltpu.sync_copy(head_hbm.at[wid, sc], hv)
        pltpu.sync_copy(tail_hbm.at[wid, sc], idx)
        pltpu.sync_copy(ktype_hbm.at[wid, sc], kv)
        pltpu.sync_copy(exps_hbm.at[wid, sc], wv)

        def _prep(j, cy):
            for i in range(_BP // 16):
                sl = pl.ds(i * 16, 16)
                idx[j, sl] = (kv[j, sl] - 1) * _NC + idx[j, sl]
                zg = plsc.load_gather(zv, [hv[j, sl]])
                wv[j, sl] = wv[j, sl] / (zg + 1e-16)
            return cy
        lax.fori_loop(0, _NS2, _prep, 0)

        pltpu.async_copy(tflat_hbm.at[idx.at[0]], rows.at[0], gsem)

        def _main(j, cy):
            slot = lax.rem(j, 2)
            nslot = lax.rem(j + 1, 2)
            pltpu.make_async_copy(
                tflat_hbm.at[pl.ds(0, _BP)], rows.at[0], gsem).wait()

            def _scale(e, cy2):
                wsp = plsc.load_gather(
                    wv, [jnp.full((16,), j, jnp.int32),
                         jnp.full((16,), e, jnp.int32)])
                for i in range(_D // 16):
                    sl = pl.ds(i * 16, 16)
                    rows[slot, e, sl] = rows[slot, e, sl] * wsp
                return cy2
            lax.fori_loop(0, _BP, _scale, 0)

            pltpu.async_copy(rows.at[slot], cat_s.at[hv.at[j]], ssem, add=True)

            @pl.when(j + 1 < _NS2)
            def _():
                @pl.when(j >= 1)
                def _():
                    pltpu.make_async_copy(
                        tflat_hbm.at[pl.ds(0, _BP)], rows.at[0], ssem).wait()
                pltpu.async_copy(
                    tflat_hbm.at[idx.at[j + 1]], rows.at[nslot], gsem)
            return cy
        lax.fori_loop(0, _NS2, _main, 0)
        pltpu.make_async_copy(
            tflat_hbm.at[pl.ds(0, _BP)], rows.at[0], ssem).wait()
        pltpu.make_async_copy(
            tflat_hbm.at[pl.ds(0, _BP)], rows.at[0], ssem).wait()
        return cy0
    lax.fori_loop(0, _SC2, _super, 0)
    plsc.subcore_barrier()

    def _out(i, carry):
        r0 = s * _STRIPE + i * _BP
        pltpu.sync_copy(cat_s.at[pl.ds(r0, _BP)], rows.at[0])
        pltpu.sync_copy(rows.at[0], cat_hbm.at[c, pl.ds(r0, _BP)])
        return carry
    lax.fori_loop(0, _STRIPE // _BP, _out, 0)


def kernel(category_emb, user_emb, edge_index, edge_type, interact_mat, weight):
    npad = _EP - _E
    padh = _NC + (jnp.arange(npad, dtype=jnp.int32) % (_NZ - _NC))
    head = jnp.concatenate([edge_index[0].astype(jnp.int32), padh])
    tail = jnp.concatenate(
        [edge_index[1].astype(jnp.int32), jnp.zeros((npad,), jnp.int32)])
    ktype = jnp.concatenate(
        [edge_type.astype(jnp.int32), jnp.ones((npad,), jnp.int32)])
    head1 = head.reshape(_NW, _NS1, _BP)
    tail1 = tail.reshape(_NW, _NS1, _BP)
    ktype1 = ktype.reshape(_NW, _NS1, _BP)
    head2 = head.reshape(_NW, _SC2, _NS2, _BP)
    tail2 = tail.reshape(_NW, _SC2, _NS2, _BP)
    ktype2 = ktype.reshape(_NW, _SC2, _NS2, _BP)
    wpad = jnp.pad(weight, ((0, _RP - _R), (0, 0)))

    a, t = _make_tables(category_emb, wpad)
    aflat = jnp.pad(a.reshape(_NC * _RP), (0, (_NZ - _NC) * _RP))
    tflat = t.reshape(_R * _NC, _D)

    exps, zpart = _s1(head1, tail1, ktype1, aflat)
    exps2 = exps.reshape(_NW, _SC2, _NS2, _BP)
    (catpart,) = _s2(head2, tail2, ktype2, exps2, zpart, tflat)
    category_agg = _make_combine(catpart)

    user_agg = _make_user(interact_mat, category_emb, user_emb, wpad)
    return (category_agg, user_agg)


# speedup vs baseline: 9.5043x; 1.2396x over previous
import functools

import jax
import jax.numpy as jnp
from jax import lax
from jax.experimental import pallas as pl
from jax.experimental.pallas import tpu as pltpu
from jax.experimental.pallas import tpu_sc as plsc

_NC = 10000
_NU = 4096
_D = 128
_E = 320000
_R = 15
_RP = 16

_NW = 32
_BP = 128
_PW = 10240
_EP = _NW * _PW
_NS1 = _PW // _BP
_SC2 = 10
_B2 = 64
_NS2 = _PW // (_SC2 * _B2)
_RING = 4
_NZ = 10240
_STRIPE = _NZ // 16


def _tables_body(emb_ref, w_ref, a_ref, t_ref):
    k = pl.program_id(0)

    @pl.when(k == 0)
    def _():
        e2 = emb_ref[...] * emb_ref[...]
        w2 = w_ref[...] * w_ref[...]
        a_ref[...] = lax.dot_general(
            e2, w2, (((1,), (1,)), ((), ())),
            preferred_element_type=jnp.float32)

    wrow = w_ref[pl.ds(k, 1), :]
    t_ref[...] = (emb_ref[...] * wrow)[None]


def _make_tables(emb, wpad):
    return pl.pallas_call(
        _tables_body,
        grid=(_R,),
        in_specs=[
            pl.BlockSpec((_NC, _D), lambda k: (0, 0)),
            pl.BlockSpec((_RP, _D), lambda k: (0, 0)),
        ],
        out_specs=[
            pl.BlockSpec((_NC, _RP), lambda k: (0, 0)),
            pl.BlockSpec((1, _NC, _D), lambda k: (k, 0, 0)),
        ],
        out_shape=[
            jax.ShapeDtypeStruct((_NC, _RP), jnp.float32),
            jax.ShapeDtypeStruct((_R, _NC, _D), jnp.float32),
        ],
    )(emb, wpad)


def _user_body(im_ref, emb_ref, ue_ref, w_ref, out_ref):
    agg = lax.dot_general(
        im_ref[...], emb_ref[...], (((1,), (0,)), ((), ())),
        preferred_element_type=jnp.float32)
    logits = lax.dot_general(
        ue_ref[...], w_ref[...], (((1,), (1,)), ((), ())),
        preferred_element_type=jnp.float32)
    col = lax.broadcasted_iota(jnp.int32, logits.shape, 1)
    logits = jnp.where(col < _R, logits, -1e30)
    m = jnp.max(logits, axis=-1, keepdims=True)
    ex = jnp.exp(logits - m)
    score = ex / jnp.sum(ex, axis=-1, keepdims=True)
    corr = lax.dot_general(
        score, w_ref[...], (((1,), (0,)), ((), ())),
        preferred_element_type=jnp.float32)
    out_ref[...] = agg + corr * agg


def _make_user(interact_mat, emb, user_emb, wpad):
    bu = 256
    return pl.pallas_call(
        _user_body,
        grid=(_NU // bu,),
        in_specs=[
            pl.BlockSpec((bu, _NC), lambda i: (i, 0)),
            pl.BlockSpec((_NC, _D), lambda i: (0, 0)),
            pl.BlockSpec((bu, _D), lambda i: (i, 0)),
            pl.BlockSpec((_RP, _D), lambda i: (0, 0)),
        ],
        out_specs=pl.BlockSpec((bu, _D), lambda i: (i, 0)),
        out_shape=jax.ShapeDtypeStruct((_NU, _D), jnp.float32),
    )(interact_mat, emb, user_emb, wpad)


def _combine_body(in_ref, out_ref):
    out_ref[...] = in_ref[0] + in_ref[1]


def _make_combine(catpart):
    br = 1000
    return pl.pallas_call(
        _combine_body,
        grid=(_NC // br,),
        in_specs=[pl.BlockSpec((2, br, _D), lambda i: (0, i, 0))],
        out_specs=pl.BlockSpec((br, _D), lambda i: (i, 0)),
        out_shape=jax.ShapeDtypeStruct((_NC, _D), jnp.float32),
    )(catpart)


@functools.partial(
    pl.kernel,
    mesh=plsc.VectorSubcoreMesh(core_axis_name="c", subcore_axis_name="s"),
    compiler_params=pltpu.CompilerParams(needs_layout_passes=False),
    out_type=[
        jax.ShapeDtypeStruct((_NW, _NS1, _BP), jnp.float32),
        jax.ShapeDtypeStruct((2, _NZ), jnp.float32),
    ],
    scratch_types=[
        pltpu.VMEM((_NS1, _BP), jnp.int32),
        pltpu.VMEM((_NS1, _BP), jnp.int32),
        pltpu.VMEM((_NS1, _BP), jnp.int32),
        pltpu.VMEM((_NS1, _BP), jnp.int32),
        pltpu.VMEM((_NS1, _BP), jnp.int32),
        pltpu.VMEM((_NS1, _BP), jnp.float32),
        pltpu.VMEM((_NS1, _BP), jnp.float32),
        pltpu.VMEM((_STRIPE,), jnp.float32),
        pltpu.VMEM_SHARED((_NZ,), jnp.float32),
        pltpu.SemaphoreType.DMA,
        pltpu.SemaphoreType.DMA,
    ],
)
def _s1(head_hbm, tail_hbm, ktype_hbm, aflat_hbm, exps_hbm, zpart_hbm,
        hv, tv, kv, ih, it, ah, at, zb, zs, gsem, ssem):
    c = lax.axis_index("c")
    s = lax.axis_index("s")
    wid = s * 2 + c

    def _zfill(i, carry):
        zb[pl.ds(i * 16, 16)] = jnp.zeros((16,), jnp.float32)
        return carry
    lax.fori_loop(0, _STRIPE // 16, _zfill, 0)
    pltpu.sync_copy(zb, zs.at[pl.ds(s * _STRIPE, _STRIPE)])
    plsc.subcore_barrier()

    pltpu.sync_copy(head_hbm.at[wid], hv)
    pltpu.sync_copy(tail_hbm.at[wid], tv)
    pltpu.sync_copy(ktype_hbm.at[wid], kv)

    def _idx(j, cy):
        for i in range(_BP // 16):
            sl = pl.ds(i * 16, 16)
            k16 = kv[j, sl] - 1
            ih[j, sl] = hv[j, sl] * _RP + k16
            it[j, sl] = tv[j, sl] * _RP + k16
        return cy
    lax.fori_loop(0, _NS1, _idx, 0)

    def _fire(j, cy):
        pltpu.async_copy(aflat_hbm.at[ih.at[j]], ah.at[j], gsem)
        pltpu.async_copy(aflat_hbm.at[it.at[j]], at.at[j], gsem)
        return cy
    lax.fori_loop(0, _NS1, _fire, 0)
    pltpu.make_async_copy(exps_hbm.at[0], ah, gsem).wait()
    pltpu.make_async_copy(exps_hbm.at[0], at, gsem).wait()

    def _exp(j, cy):
        for i in range(_BP // 16):
            sl = pl.ds(i * 16, 16)
            ah[j, sl] = jnp.exp(ah[j, sl] * at[j, sl])
        return cy
    lax.fori_loop(0, _NS1, _exp, 0)

    pltpu.sync_copy(ah, exps_hbm.at[wid])

    def _scat(j, cy):
        pltpu.async_copy(ah.at[j], zs.at[hv.at[j]], ssem, add=True)
        return cy
    lax.fori_loop(0, _NS1, _scat, 0)
    pltpu.make_async_copy(exps_hbm.at[0], ah, ssem).wait()
    plsc.subcore_barrier()

    pltpu.sync_copy(zs.at[pl.ds(s * _STRIPE, _STRIPE)], zb)
    pltpu.sync_copy(zb, zpart_hbm.at[c, pl.ds(s * _STRIPE, _STRIPE)])


@functools.partial(
    pl.kernel,
    mesh=plsc.VectorSubcoreMesh(core_axis_name="c", subcore_axis_name="s"),
    compiler_params=pltpu.CompilerParams(needs_layout_passes=False),
    out_type=[
        jax.ShapeDtypeStruct((2, _NZ, _D), jnp.float32),
    ],
    scratch_types=[
        pltpu.VMEM((_NS2, _B2), jnp.int32),
        pltpu.VMEM((_NS2 * _B2,), jnp.int32),
        pltpu.VMEM((_NS2 * _B2,), jnp.int32),
        pltpu.VMEM((_NS2 * _B2,), jnp.float32),
        pltpu.VMEM((_NZ,), jnp.float32),
        pltpu.VMEM((_RING, _B2, _D), jnp.float32),
        pltpu.VMEM_SHARED((_NZ, _D), jnp.float32),
        pltpu.SemaphoreType.DMA,
        pltpu.SemaphoreType.DMA,
    ],
)
def _s2(head2_hbm, tail_hbm, ktype_hbm, exps_hbm, zpart_hbm,
        tflat_hbm, cat_hbm, hv, kv, idx, wv, zv, rows, cat_s, gsem, ssem):
    c = lax.axis_index("c")
    s = lax.axis_index("s")
    wid = s * 2 + c

    def _zrow(i, carry):
        for j in range(_D // 16):
            rows[0, i, pl.ds(j * 16, 16)] = jnp.zeros((16,), jnp.float32)
        return carry
    lax.fori_loop(0, _B2, _zrow, 0)

    def _zcat(i, carry):
        pltpu.sync_copy(rows.at[0],
                        cat_s.at[pl.ds(s * _STRIPE + i * _B2, _B2)])
        return carry
    lax.fori_loop(0, _STRIPE // _B2, _zcat, 0)
    plsc.subcore_barrier()

    pltpu.sync_copy(zpart_hbm.at[0], zv)

    def _zld(i, carry):
        pltpu.sync_copy(zpart_hbm.at[1, pl.ds(i * 1024, 1024)], wv)

        def _za(g, c2):
            zv[pl.ds(i * 1024 + g * 16, 16)] = (
                zv[pl.ds(i * 1024 + g * 16, 16)] + wv[pl.ds(g * 16, 16)])
            return c2
        lax.fori_loop(0, 1024 // 16, _za, 0)
        return carry
    lax.fori_loop(0, _NZ // 1024, _zld, 0)

    def _super(sc, cy0):
        pltpu.sync_copy(head2_hbm.at[wid, sc], hv)
        pltpu.sync_copy(tail_hbm.at[wid, sc], idx)
        pltpu.sync_copy(ktype_hbm.at[wid, sc], kv)
        pltpu.sync_copy(exps_hbm.at[wid, sc], wv)

        def _prep(j, cy):
            for i in range(_B2 // 16):
                fl = pl.ds(j * _B2 + i * 16, 16)
                sl = pl.ds(i * 16, 16)
                idx[fl] = (kv[fl] - 1) * _NC + idx[fl]
                zg = plsc.load_gather(zv, [hv[j, sl]])
                wv[fl] = wv[fl] / (zg + 1e-16)
            return cy
        lax.fori_loop(0, _NS2, _prep, 0)

        pltpu.async_copy(
            tflat_hbm.at[idx.at[pl.ds(0, _B2)]], rows.at[0], gsem)
        pltpu.async_copy(
            tflat_hbm.at[idx.at[pl.ds(_B2, _B2)]], rows.at[1], gsem)

        def _main(j, cy):
            slot = lax.rem(j, _RING)

            @pl.when(j + 2 < _NS2)
            def _():
                @pl.when(j >= 2)
                def _():
                    pltpu.make_async_copy(
                        tflat_hbm.at[pl.ds(0, _B2)], rows.at[0], ssem).wait()
                pltpu.async_copy(
                    tflat_hbm.at[idx.at[pl.ds((j + 2) * _B2, _B2)]],
                    rows.at[lax.rem(j + 2, _RING)], gsem)

            pltpu.make_async_copy(
                tflat_hbm.at[pl.ds(0, _B2)], rows.at[0], gsem).wait()

            def _scale(e, cy2):
                wsp = plsc.load_gather(
                    wv, [jnp.full((16,), j * _B2, jnp.int32) + e])
                for i in range(_D // 16):
                    sl = pl.ds(i * 16, 16)
                    rows[slot, e, sl] = rows[slot, e, sl] * wsp
                return cy2
            lax.fori_loop(0, _B2, _scale, 0)

            pltpu.async_copy(rows.at[slot], cat_s.at[hv.at[j]], ssem, add=True)
            return cy
        lax.fori_loop(0, _NS2, _main, 0)
        for _i in range(_RING):
            pltpu.make_async_copy(
                tflat_hbm.at[pl.ds(0, _B2)], rows.at[0], ssem).wait()
        return cy0
    lax.fori_loop(0, _SC2, _super, 0)
    plsc.subcore_barrier()

    def _out(i, carry):
        r0 = s * _STRIPE + i * _B2
        pltpu.sync_copy(cat_s.at[pl.ds(r0, _B2)], rows.at[0])
        pltpu.sync_copy(rows.at[0], cat_hbm.at[c, pl.ds(r0, _B2)])
        return carry
    lax.fori_loop(0, _STRIPE // _B2, _out, 0)


def kernel(category_emb, user_emb, edge_index, edge_type, interact_mat, weight):
    npad = _PW - _E // _NW
    padh = jnp.broadcast_to(
        _NC + jnp.arange(npad, dtype=jnp.int32), (_NW, npad))
    head = jnp.concatenate(
        [edge_index[0].astype(jnp.int32).reshape(_NW, -1), padh], axis=1)
    tail = jnp.concatenate(
        [edge_index[1].astype(jnp.int32).reshape(_NW, -1),
         jnp.zeros((_NW, npad), jnp.int32)], axis=1)
    ktype = jnp.concatenate(
        [edge_type.astype(jnp.int32).reshape(_NW, -1),
         jnp.ones((_NW, npad), jnp.int32)], axis=1)
    head1 = head.reshape(_NW, _NS1, _BP)
    tail1 = tail.reshape(_NW, _NS1, _BP)
    ktype1 = ktype.reshape(_NW, _NS1, _BP)
    head2 = head.reshape(_NW, _SC2, _NS2, _B2)
    tail2 = tail.reshape(_NW, _SC2, _NS2 * _B2)
    ktype2 = ktype.reshape(_NW, _SC2, _NS2 * _B2)
    wpad = jnp.pad(weight, ((0, _RP - _R), (0, 0)))

    a, t = _make_tables(category_emb, wpad)
    aflat = jnp.pad(a.reshape(_NC * _RP), (0, (_NZ - _NC) * _RP))
    tflat = t.reshape(_R * _NC, _D)

    exps, zpart = _s1(head1, tail1, ktype1, aflat)
    exps2 = exps.reshape(_NW, _SC2, _NS2 * _B2)
    (catpart,) = _s2(head2, tail2, ktype2, exps2, zpart, tflat)
    category_agg = _make_combine(catpart)

    user_agg = _make_user(interact_mat, category_emb, user_emb, wpad)
    return (category_agg, user_agg)
